# Initial kernel scaffold; baseline (speedup 1.0000x reference)
import jax
import jax.numpy as jnp
from jax.experimental import pallas as pl


def kernel(x, encoding):
    raise NotImplementedError("write your pallas kernel here")


# speedup vs baseline: 3.1402x; 3.1402x over previous
import functools

import jax
import jax.numpy as jnp
from jax import lax
from jax.experimental import pallas as pl
from jax.experimental.pallas import tpu as pltpu
from jax.experimental.pallas import tpu_sc as plsc

EMB = 128


@functools.cache
def _build_gather(N):
    info = plsc.get_sparse_core_info()
    NC, NS = info.num_cores, info.num_subcores
    NW = NC * NS
    rows_per_w = N // NW
    C = 256
    K = C // 128
    G = rows_per_w // C
    assert N % NW == 0 and rows_per_w % C == 0

    mesh = plsc.VectorSubcoreMesh(core_axis_name="c", subcore_axis_name="s")

    @functools.partial(
        pl.kernel,
        mesh=mesh,
        out_type=jax.ShapeDtypeStruct((N, EMB), jnp.float32),
        scratch_types=[
            pltpu.VMEM((K, 128), jnp.int32),
            pltpu.VMEM((C, EMB), jnp.float32),
            pltpu.SemaphoreType.DMA,
        ],
    )
    def gather_kernel(idx_hbm, table_hbm, out_hbm, idx_v, rows_v, sem):
        wid = lax.axis_index("s") * NC + lax.axis_index("c")
        row0 = wid * (rows_per_w // 128)

        def body(g, carry):
            off = wid * rows_per_w + g * C
            pltpu.sync_copy(idx_hbm.at[pl.ds(row0 + g * K, K)], idx_v)
            copies = [
                pltpu.async_copy(
                    table_hbm.at[idx_v.at[j]],
                    rows_v.at[pl.ds(j * 128, 128)],
                    sem,
                )
                for j in range(K)
            ]
            for c in copies:
                c.wait()
            pltpu.sync_copy(rows_v, out_hbm.at[pl.ds(off, C)])
            return carry

        lax.fori_loop(0, G, body, 0)

    return gather_kernel


def kernel(x, encoding):
    B, T = x.shape
    N = B * T
    xf = x.reshape(N // 128, 128).astype(jnp.int32)
    out = _build_gather(N)(xf, encoding)
    return out.reshape(B, T, EMB)


# speedup vs baseline: 3.4596x; 1.1017x over previous
import functools

import jax
import jax.numpy as jnp
from jax import lax
from jax.experimental import pallas as pl
from jax.experimental.pallas import tpu as pltpu
from jax.experimental.pallas import tpu_sc as plsc

EMB = 128


@functools.cache
def _build_gather(N):
    info = plsc.get_sparse_core_info()
    NC, NS = info.num_cores, info.num_subcores
    NW = NC * NS
    rows_per_w = N // NW
    C = 256
    K = C // 128
    G = rows_per_w // C
    IR = rows_per_w // 128
    assert N % NW == 0 and rows_per_w % C == 0 and G % 2 == 0 and G >= 4

    mesh = plsc.VectorSubcoreMesh(core_axis_name="c", subcore_axis_name="s")

    @functools.partial(
        pl.kernel,
        mesh=mesh,
        out_type=jax.ShapeDtypeStruct((N, EMB), jnp.float32),
        scratch_types=[
            pltpu.VMEM((IR, 128), jnp.int32),
            pltpu.VMEM((2, C, EMB), jnp.float32),
            pltpu.SemaphoreType.DMA,
            pltpu.SemaphoreType.DMA,
            pltpu.SemaphoreType.DMA,
            pltpu.SemaphoreType.DMA,
        ],
    )
    def gather_kernel(idx_hbm, table_hbm, out_hbm, idx_v, rows_v,
                      gsem0, gsem1, osem0, osem1):
        gsem = (gsem0, gsem1)
        osem = (osem0, osem1)
        wid = lax.axis_index("s") * NC + lax.axis_index("c")
        base = wid * rows_per_w
        pltpu.sync_copy(idx_hbm.at[pl.ds(wid * IR, IR)], idx_v)

        def start_gather(c, u):
            for j in range(K):
                pltpu.async_copy(
                    table_hbm.at[idx_v.at[c * K + j]],
                    rows_v.at[u, pl.ds(j * 128, 128)],
                    gsem[u],
                )

        def wait_gather(u):
            pltpu.make_async_copy(
                out_hbm.at[pl.ds(0, C)], rows_v.at[u], gsem[u]
            ).wait()

        def start_store(c, u):
            pltpu.async_copy(
                rows_v.at[u], out_hbm.at[pl.ds(base + c * C, C)], osem[u]
            )

        def wait_store(u):
            pltpu.make_async_copy(
                rows_v.at[u], out_hbm.at[pl.ds(base, C)], osem[u]
            ).wait()

        start_gather(0, 0)
        wait_gather(0)
        start_store(0, 0)
        start_gather(1, 1)

        def body(t, carry):
            for i in range(2):
                c = 1 + 2 * t + i
                u = (1 - i)
                wait_gather(u)
                start_store(c, u)
                wait_store(1 - u)
                start_gather(c + 1, 1 - u)
            return carry

        lax.fori_loop(0, (G - 2) // 2, body, 0)

        wait_gather(1)
        start_store(G - 1, 1)
        wait_store(0)
        wait_store(1)

    return gather_kernel


def kernel(x, encoding):
    B, T = x.shape
    N = B * T
    xf = x.reshape(N // 128, 128).astype(jnp.int32)
    out = _build_gather(N)(xf, encoding)
    return out.reshape(B, T, EMB)


# speedup vs baseline: 5.4146x; 1.5651x over previous
import functools

import jax
import jax.numpy as jnp
from jax import lax
from jax.experimental import pallas as pl
from jax.experimental.pallas import tpu as pltpu
from jax.experimental.pallas import tpu_sc as plsc

EMB = 128


@functools.cache
def _build_gather(B, T):
    info = plsc.get_sparse_core_info()
    NC, NS = info.num_cores, info.num_subcores
    NW = NC * NS
    b_per_w = B // NW
    NB = 2
    G = b_per_w // NB
    assert B % NW == 0 and b_per_w % NB == 0 and G % 2 == 0 and G >= 4

    mesh = plsc.VectorSubcoreMesh(core_axis_name="c", subcore_axis_name="s")

    @functools.partial(
        pl.kernel,
        mesh=mesh,
        out_type=jax.ShapeDtypeStruct((B, T, EMB), jnp.float32),
        scratch_types=[
            pltpu.VMEM((2, NB, T), jnp.int32),
            pltpu.VMEM((2, NB, T, EMB), jnp.float32),
            pltpu.SemaphoreType.DMA,
            pltpu.SemaphoreType.DMA,
            pltpu.SemaphoreType.DMA,
            pltpu.SemaphoreType.DMA,
            pltpu.SemaphoreType.DMA,
            pltpu.SemaphoreType.DMA,
        ],
    )
    def gather_kernel(x_hbm, table_hbm, out_hbm, idx_v, rows_v,
                      gsem0, gsem1, osem0, osem1, isem0, isem1):
        gsem = (gsem0, gsem1)
        osem = (osem0, osem1)
        isem = (isem0, isem1)
        wid = lax.axis_index("s") * NC + lax.axis_index("c")
        base = wid * b_per_w

        def start_idx(c, u):
            pltpu.async_copy(
                x_hbm.at[pl.ds(base + c * NB, NB)], idx_v.at[u], isem[u]
            )

        def wait_idx(u):
            pltpu.make_async_copy(
                x_hbm.at[pl.ds(base, NB)], idx_v.at[u], isem[u]
            ).wait()

        def start_gather(c, u):
            for j in range(NB):
                pltpu.async_copy(
                    table_hbm.at[idx_v.at[u, j]],
                    rows_v.at[u, j],
                    gsem[u],
                )

        def wait_gather(u):
            pltpu.make_async_copy(
                out_hbm.at[pl.ds(0, NB)], rows_v.at[u], gsem[u]
            ).wait()

        def start_store(c, u):
            pltpu.async_copy(
                rows_v.at[u], out_hbm.at[pl.ds(base + c * NB, NB)], osem[u]
            )

        def wait_store(u):
            pltpu.make_async_copy(
                rows_v.at[u], out_hbm.at[pl.ds(base, NB)], osem[u]
            ).wait()

        start_idx(0, 0)
        start_idx(1, 1)
        wait_idx(0)
        start_gather(0, 0)
        wait_gather(0)
        start_idx(2, 0)
        start_store(0, 0)
        wait_idx(1)
        start_gather(1, 1)

        def body(t, carry):
            for i in range(2):
                c = 1 + 2 * t + i
                u = 1 - i
                wait_gather(u)
                start_idx(c + 2, u)
                start_store(c, u)
                wait_store(1 - u)
                wait_idx(1 - u)
                start_gather(c + 1, 1 - u)
            return carry

        lax.fori_loop(0, (G - 2) // 2 - 1, body, 0)

        for i in range(2):
            c = G - 3 + i
            u = 1 - i
            wait_gather(u)
            if c + 2 < G:
                start_idx(c + 2, u)
            start_store(c, u)
            wait_store(1 - u)
            wait_idx(1 - u)
            start_gather(c + 1, 1 - u)

        wait_gather(1)
        start_store(G - 1, 1)
        wait_store(0)
        wait_store(1)

    return gather_kernel


def kernel(x, encoding):
    B, T = x.shape
    out = _build_gather(B, T)(x.astype(jnp.int32), encoding)
    return out
